# Initial kernel scaffold; baseline (speedup 1.0000x reference)
import jax
import jax.numpy as jnp
from jax.experimental import pallas as pl


def kernel(x, edge_index, batch, W1, b1, Ws1, bs1, W2, b2, Ws2, bs2, W3, b3, Ws3, bs3, Wl, bl):
    raise NotImplementedError("write your pallas kernel here")


# speedup vs baseline: 17.3759x; 17.3759x over previous
import functools
import jax
import jax.numpy as jnp
from jax import lax
from jax.experimental import pallas as pl
from jax.experimental.pallas import tpu as pltpu
from jax.experimental.pallas import tpu_sc as plsc

N = 10000
F = 128
G = 64
RATIO = 0.8
NC, NS, L = 2, 16, 16
NW = NC * NS
NP = 10112
ROWS = NP // 128
DUMMY = N
E = 320000
ET = NP
EP = NW * ET
NVEC = NP // L

_mesh = plsc.VectorSubcoreMesh(core_axis_name="c", subcore_axis_name="s")
_sc_params = pltpu.CompilerParams(needs_layout_passes=False)


@functools.partial(
    pl.kernel, mesh=_mesh, compiler_params=_sc_params,
    out_type=(jax.ShapeDtypeStruct((NW, ET), jnp.int32),
              jax.ShapeDtypeStruct((NW, NP), jnp.float32)),
    scratch_types=[pltpu.VMEM((ET,), jnp.int32),
                   pltpu.VMEM((ET,), jnp.int32),
                   pltpu.VMEM((NP,), jnp.int32),
                   pltpu.VMEM((ET,), jnp.int32),
                   pltpu.VMEM((NP,), jnp.float32)],
)
def _sc_deg(src_hbm, dst_hbm, keep_hbm, ndst_hbm, degp_hbm,
            src_v, dst_v, keep_v, ndst_v, acc_v):
    w = lax.axis_index("s") * NC + lax.axis_index("c")
    pltpu.sync_copy(src_hbm.at[w], src_v)
    pltpu.sync_copy(dst_hbm.at[w], dst_v)
    pltpu.sync_copy(keep_hbm, keep_v)

    def zero(i, carry):
        acc_v[pl.ds(i * L, L)] = jnp.zeros((L,), jnp.float32)
        return carry
    lax.fori_loop(0, NVEC, zero, 0)

    ones = jnp.ones((L,), jnp.float32)

    def body(i, carry):
        vs = src_v[pl.ds(i * L, L)]
        vd = dst_v[pl.ds(i * L, L)]
        ks = plsc.load_gather(keep_v, [vs])
        kd = plsc.load_gather(keep_v, [vd])
        nd = jnp.where((ks > 0) & (kd > 0), vd, DUMMY)
        ndst_v[pl.ds(i * L, L)] = nd
        plsc.addupdate_scatter(acc_v, [nd], ones)
        return carry
    lax.fori_loop(0, NVEC, body, 0)

    pltpu.sync_copy(ndst_v, ndst_hbm.at[w])
    pltpu.sync_copy(acc_v, degp_hbm.at[w])


@functools.partial(
    pl.kernel, mesh=_mesh, compiler_params=_sc_params,
    out_type=jax.ShapeDtypeStruct((NC, NP, F), jnp.float32),
    scratch_types=[pltpu.VMEM((ROWS, 128), jnp.int32),
                   pltpu.VMEM((ROWS, 128), jnp.int32),
                   pltpu.VMEM((128, F), jnp.float32),
                   pltpu.VMEM_SHARED((NP, F), jnp.float32),
                   pltpu.SemaphoreType.DMA],
)
def _sc_rowagg(g_hbm, src_hbm, dst_hbm, zeros_hbm, part_hbm,
               src_v, dst_v, rows_v, acc_sh, sem):
    c = lax.axis_index("c")
    s = lax.axis_index("s")
    w = s * NC + c
    pltpu.sync_copy(src_hbm.at[w], src_v)
    pltpu.sync_copy(dst_hbm.at[w], dst_v)
    pltpu.sync_copy(zeros_hbm, acc_sh.at[pl.ds(s * (NP // NS), NP // NS)])
    plsc.subcore_barrier()

    def body(j, carry):
        pltpu.async_copy(g_hbm.at[src_v.at[j]], rows_v, sem).wait()
        pltpu.sync_copy(rows_v, acc_sh.at[dst_v.at[j]], add=True)
        return carry
    lax.fori_loop(0, ROWS, body, 0)

    plsc.subcore_barrier()
    stripe = pl.ds(s * (NP // NS), NP // NS)
    pltpu.sync_copy(acc_sh.at[stripe], part_hbm.at[c, stripe])


@functools.partial(
    pl.kernel, mesh=_mesh, compiler_params=_sc_params,
    out_type=jax.ShapeDtypeStruct((NW, NP), jnp.float32),
    scratch_types=[pltpu.VMEM((ET,), jnp.int32),
                   pltpu.VMEM((ET,), jnp.int32),
                   pltpu.VMEM((NP,), jnp.float32),
                   pltpu.VMEM((NP,), jnp.float32)],
)
def _sc_scalagg(gs_hbm, src_hbm, dst_hbm, sagg_hbm,
                src_v, dst_v, gs_v, acc_v):
    w = lax.axis_index("s") * NC + lax.axis_index("c")
    pltpu.sync_copy(src_hbm.at[w], src_v)
    pltpu.sync_copy(dst_hbm.at[w], dst_v)
    pltpu.sync_copy(gs_hbm, gs_v)

    def zero(i, carry):
        acc_v[pl.ds(i * L, L)] = jnp.zeros((L,), jnp.float32)
        return carry
    lax.fori_loop(0, NVEC, zero, 0)

    def body(i, carry):
        vs = src_v[pl.ds(i * L, L)]
        vd = dst_v[pl.ds(i * L, L)]
        val = plsc.load_gather(gs_v, [vs])
        plsc.addupdate_scatter(acc_v, [vd], val)
        return carry
    lax.fori_loop(0, NVEC, body, 0)

    pltpu.sync_copy(acc_v, sagg_hbm.at[w])


def _tc1_body(x_ref, w_ref, degp_ref, nm_ref, g_ref, st_ref):
    deg = jnp.sum(degp_ref[...], axis=0) + nm_ref[0, 0, :]
    dinv = jnp.where(deg > 0, lax.rsqrt(deg), 0.0)
    h_pre = jnp.dot(x_ref[...], w_ref[...], preferred_element_type=jnp.float32)
    g_ref[...] = h_pre * dinv[:, None]
    st_ref[...] = h_pre * (nm_ref[0, 0, :] * dinv * dinv)[:, None]


def _tc1(x, W, degp, nm3):
    return pl.pallas_call(
        _tc1_body,
        grid=(ROWS,),
        in_specs=[pl.BlockSpec((128, F), lambda i: (i, 0)),
                  pl.BlockSpec((F, F), lambda i: (0, 0)),
                  pl.BlockSpec((NW, 128), lambda i: (0, i)),
                  pl.BlockSpec((1, 1, 128), lambda i: (i, 0, 0))],
        out_specs=[pl.BlockSpec((128, F), lambda i: (i, 0)),
                   pl.BlockSpec((128, F), lambda i: (i, 0))],
        out_shape=[jax.ShapeDtypeStruct((NP, F), jnp.float32),
                   jax.ShapeDtypeStruct((NP, F), jnp.float32)],
    )(x, W, degp, nm3)


def _tc2_body(part_ref, st_ref, degp_ref, nm_ref, b_ref, ws_ref,
              h_ref, spre_ref, gs_ref):
    deg = jnp.sum(degp_ref[...], axis=0) + nm_ref[0, 0, :]
    dinv = jnp.where(deg > 0, lax.rsqrt(deg), 0.0)
    agg = part_ref[0] + part_ref[1]
    h = jnp.maximum(agg * dinv[:, None] + st_ref[...] + b_ref[0, 0, :][None, :], 0.0)
    h_ref[...] = h
    spre = jnp.sum(h * ws_ref[0, 0, :][None, :], axis=1)
    spre_ref[0, 0, :] = spre
    gs_ref[0, 0, :] = spre * dinv


def _tc2(part, st, degp, nm3, b3, ws3):
    return pl.pallas_call(
        _tc2_body,
        grid=(ROWS,),
        in_specs=[pl.BlockSpec((NC, 128, F), lambda i: (0, i, 0)),
                  pl.BlockSpec((128, F), lambda i: (i, 0)),
                  pl.BlockSpec((NW, 128), lambda i: (0, i)),
                  pl.BlockSpec((1, 1, 128), lambda i: (i, 0, 0)),
                  pl.BlockSpec((1, 1, 128), lambda i: (0, 0, 0)),
                  pl.BlockSpec((1, 1, 128), lambda i: (0, 0, 0))],
        out_specs=[pl.BlockSpec((128, F), lambda i: (i, 0)),
                   pl.BlockSpec((1, 1, 128), lambda i: (i, 0, 0)),
                   pl.BlockSpec((1, 1, 128), lambda i: (i, 0, 0))],
        out_shape=[jax.ShapeDtypeStruct((NP, F), jnp.float32),
                   jax.ShapeDtypeStruct((ROWS, 1, 128), jnp.float32),
                   jax.ShapeDtypeStruct((ROWS, 1, 128), jnp.float32)],
    )(part, st, degp, nm3, b3, ws3)


def _tc3_body(sagg_ref, degp_ref, nm_ref, spre_ref, bs_ref, score_ref):
    deg = jnp.sum(degp_ref[...], axis=0) + nm_ref[0, 0, :]
    dinv = jnp.where(deg > 0, lax.rsqrt(deg), 0.0)
    sa = jnp.sum(sagg_ref[...], axis=0)
    score_ref[0, 0, :] = (sa * dinv
                          + spre_ref[0, 0, :] * (nm_ref[0, 0, :] * dinv * dinv)
                          + bs_ref[0, 0, :])


def _tc3(sagg, degp, nm3, spre3, bs3):
    return pl.pallas_call(
        _tc3_body,
        grid=(ROWS,),
        in_specs=[pl.BlockSpec((NW, 128), lambda i: (0, i)),
                  pl.BlockSpec((NW, 128), lambda i: (0, i)),
                  pl.BlockSpec((1, 1, 128), lambda i: (i, 0, 0)),
                  pl.BlockSpec((1, 1, 128), lambda i: (i, 0, 0)),
                  pl.BlockSpec((1, 1, 128), lambda i: (0, 0, 0))],
        out_specs=pl.BlockSpec((1, 1, 128), lambda i: (i, 0, 0)),
        out_shape=jax.ShapeDtypeStruct((ROWS, 1, 128), jnp.float32),
    )(sagg, degp, nm3, spre3, bs3)


def _tc4_body(h_ref, score_ref, keep_ref, batch_ref,
              hp_ref, sums_ref, cnts_ref, maxs_ref):
    i = pl.program_id(0)
    th = jnp.tanh(score_ref[0, 0, :])
    hpb = h_ref[...] * th[:, None]
    hp_ref[...] = hpb
    keep = keep_ref[0, 0, :]
    kb = jnp.where(keep > 0, batch_ref[0, 0, :], G)

    onehot = (kb[:, None] == lax.broadcasted_iota(jnp.int32, (128, G), 1)
              ).astype(jnp.float32)
    psum = lax.dot_general(onehot, hpb, (((0,), (0,)), ((), ())),
                           preferred_element_type=jnp.float32)
    pcnt = jnp.sum(onehot, axis=0)

    neg = jnp.float32(-3.0e38)
    rows = []
    for g in range(G):
        mf = (kb == g).astype(jnp.float32)[:, None]
        rows.append(jnp.max(hpb * mf + neg * (1.0 - mf), axis=0))
    pmax = jnp.stack(rows, axis=0)

    @pl.when(i == 0)
    def _():
        sums_ref[...] = jnp.zeros((G, F), jnp.float32)
        cnts_ref[...] = jnp.zeros((G, 128), jnp.float32)
        maxs_ref[...] = jnp.full((G, F), neg, jnp.float32)

    sums_ref[...] += psum
    cnts_ref[...] += pcnt[:, None]
    maxs_ref[...] = jnp.maximum(maxs_ref[...], pmax)


def _tc4(h, score3, keep3, batch3):
    return pl.pallas_call(
        _tc4_body,
        grid=(ROWS,),
        in_specs=[pl.BlockSpec((128, F), lambda i: (i, 0)),
                  pl.BlockSpec((1, 1, 128), lambda i: (i, 0, 0)),
                  pl.BlockSpec((1, 1, 128), lambda i: (i, 0, 0)),
                  pl.BlockSpec((1, 1, 128), lambda i: (i, 0, 0))],
        out_specs=[pl.BlockSpec((128, F), lambda i: (i, 0)),
                   pl.BlockSpec((G, F), lambda i: (0, 0)),
                   pl.BlockSpec((G, 128), lambda i: (0, 0)),
                   pl.BlockSpec((G, F), lambda i: (0, 0))],
        out_shape=[jax.ShapeDtypeStruct((NP, F), jnp.float32),
                   jax.ShapeDtypeStruct((G, F), jnp.float32),
                   jax.ShapeDtypeStruct((G, 128), jnp.float32),
                   jax.ShapeDtypeStruct((G, F), jnp.float32)],
    )(h, score3, keep3, batch3)


def _tc5_body(xs_ref, wl_ref, bl_ref, out_ref):
    out_ref[...] = jnp.maximum(
        jnp.dot(xs_ref[...], wl_ref[...], preferred_element_type=jnp.float32)
        + bl_ref[0][None, :], 0.0)


def _tc5(xs, Wl, bl2):
    return pl.pallas_call(
        _tc5_body,
        in_specs=[pl.BlockSpec((G, 2 * F), lambda: (0, 0)),
                  pl.BlockSpec((2 * F, F), lambda: (0, 0)),
                  pl.BlockSpec((8, F), lambda: (0, 0))],
        out_specs=pl.BlockSpec((G, F), lambda: (0, 0)),
        out_shape=jax.ShapeDtypeStruct((G, F), jnp.float32),
    )(xs, Wl, bl2)


def _topk_keep(score, valid, batch_pad):
    bmask = jnp.where(valid, batch_pad, G)
    counts = jnp.bincount(bmask, length=G + 1)
    k = jnp.ceil(RATIO * counts).astype(counts.dtype).at[G].set(0)
    order = jnp.lexsort((-score, bmask))
    bsrt = bmask[order]
    starts = jnp.cumsum(counts) - counts
    pos = jnp.arange(NP) - starts[bsrt]
    keep_sorted = pos < k[bsrt]
    return jnp.zeros((NP,), bool).at[order].set(keep_sorted)


def kernel(x, edge_index, batch, W1, b1, Ws1, bs1, W2, b2, Ws2, bs2,
           W3, b3, Ws3, bs3, Wl, bl):
    f32 = jnp.float32
    xp = jnp.zeros((NP, F), f32).at[:N, :].set(x)
    batch_pad = jnp.concatenate([batch.astype(jnp.int32),
                                 jnp.full((NP - N,), G, jnp.int32)])
    batch3 = batch_pad.reshape(ROWS, 1, 128)
    src = jnp.concatenate([edge_index[0].astype(jnp.int32),
                           jnp.zeros((EP - E,), jnp.int32)])
    dst = jnp.concatenate([edge_index[1].astype(jnp.int32),
                           jnp.full((EP - E,), DUMMY, jnp.int32)])
    src2 = src.reshape(NW, ET)
    src3 = src.reshape(NW, ROWS, 128)
    dst2 = dst.reshape(NW, ET)

    zeros_rows = jnp.zeros((NP // NS, F), f32)
    keep_i = jnp.concatenate([jnp.ones((N,), jnp.int32),
                              jnp.zeros((NP - N,), jnp.int32)])
    nm = keep_i.astype(f32)
    dst_cur2 = dst2
    xcur = xp
    xs = None
    for (W, b, Ws, bs) in ((W1, b1, Ws1, bs1), (W2, b2, Ws2, bs2),
                           (W3, b3, Ws3, bs3)):
        nm3 = nm.reshape(ROWS, 1, 128)
        b3r = b.reshape(1, 1, 128)
        ws3r = Ws.reshape(1, 1, 128)
        bs3r = jnp.broadcast_to(bs.reshape(1, 1, 1), (1, 1, 128))

        ndst2, degp = _sc_deg(src2, dst_cur2, keep_i)
        dst_cur2 = ndst2
        dst3 = ndst2.reshape(NW, ROWS, 128)

        g, st = _tc1(xcur, W, degp, nm3)
        part = _sc_rowagg(g, src3, dst3, zeros_rows)
        h, spre3, gs3 = _tc2(part, st, degp, nm3, b3r, ws3r)
        sagg = _sc_scalagg(gs3.reshape(NP), src2, ndst2)
        score3 = _tc3(sagg, degp, nm3, spre3, bs3r)

        score = score3.reshape(NP)
        keep = _topk_keep(score, nm > 0, batch_pad)
        keep_i = keep.astype(jnp.int32)
        keep3 = keep_i.astype(f32).reshape(ROWS, 1, 128)

        hp, sums, cnts, maxs = _tc4(h, score3, keep3, batch3)
        cnt = cnts[:, 0]
        gmp = jnp.where(cnt[:, None] > 0, maxs, 0.0)
        gap = sums / jnp.maximum(cnt, 1.0)[:, None]
        ro = jnp.concatenate([gmp, gap], axis=1)
        xs = ro if xs is None else xs + ro

        nm = keep_i.astype(f32)
        xcur = hp

    bl2 = jnp.broadcast_to(bl[None, :], (8, F))
    return _tc5(xs, Wl, bl2)


# speedup vs baseline: 18.1165x; 1.0426x over previous
import functools
import jax
import jax.numpy as jnp
from jax import lax
from jax.experimental import pallas as pl
from jax.experimental.pallas import tpu as pltpu
from jax.experimental.pallas import tpu_sc as plsc

N = 10000
F = 128
G = 64
RATIO = 0.8
NC, NS, L = 2, 16, 16
NW = NC * NS
NP = 10112
ROWS = NP // 128
DUMMY = N
E = 320000
ET = NP
EP = NW * ET
NVEC = NP // L
CH = 64
NCH = ET // CH

_mesh = plsc.VectorSubcoreMesh(core_axis_name="c", subcore_axis_name="s")
_sc_params = pltpu.CompilerParams(needs_layout_passes=False)


@functools.partial(
    pl.kernel, mesh=_mesh, compiler_params=_sc_params,
    out_type=(jax.ShapeDtypeStruct((NW, ET), jnp.int32),
              jax.ShapeDtypeStruct((NW, NP), jnp.float32)),
    scratch_types=[pltpu.VMEM((ET,), jnp.int32),
                   pltpu.VMEM((ET,), jnp.int32),
                   pltpu.VMEM((NP,), jnp.int32),
                   pltpu.VMEM((ET,), jnp.int32),
                   pltpu.VMEM((NP,), jnp.float32)],
)
def _sc_deg(src_hbm, dst_hbm, keep_hbm, ndst_hbm, degp_hbm,
            src_v, dst_v, keep_v, ndst_v, acc_v):
    w = lax.axis_index("s") * NC + lax.axis_index("c")
    pltpu.sync_copy(src_hbm.at[w], src_v)
    pltpu.sync_copy(dst_hbm.at[w], dst_v)
    pltpu.sync_copy(keep_hbm, keep_v)

    def zero(i, carry):
        acc_v[pl.ds(i * L, L)] = jnp.zeros((L,), jnp.float32)
        return carry
    lax.fori_loop(0, NVEC, zero, 0)

    ones = jnp.ones((L,), jnp.float32)

    def body(i, carry):
        vs = src_v[pl.ds(i * L, L)]
        vd = dst_v[pl.ds(i * L, L)]
        ks = plsc.load_gather(keep_v, [vs])
        kd = plsc.load_gather(keep_v, [vd])
        nd = jnp.where((ks > 0) & (kd > 0), vd, DUMMY)
        ndst_v[pl.ds(i * L, L)] = nd
        plsc.addupdate_scatter(acc_v, [nd], ones)
        return carry
    lax.fori_loop(0, NVEC, body, 0)

    pltpu.sync_copy(ndst_v, ndst_hbm.at[w])
    pltpu.sync_copy(acc_v, degp_hbm.at[w])


@functools.partial(
    pl.kernel, mesh=_mesh, compiler_params=_sc_params,
    out_type=jax.ShapeDtypeStruct((NC, NP, F), jnp.float32),
    scratch_types=[pltpu.VMEM((40, 128), jnp.int32),
                   pltpu.VMEM((40, 128), jnp.int32),
                   pltpu.VMEM((128, F), jnp.float32),
                   pltpu.VMEM((128, F), jnp.float32),
                   pltpu.VMEM_SHARED((NP, F), jnp.float32),
                   pltpu.SemaphoreType.DMA,
                   pltpu.SemaphoreType.DMA],
)
def _sc_rowagg(g_hbm, src_hbm, dst_hbm, zeros_hbm, part_hbm,
               src_v, dst_v, rows0_v, rows1_v, acc_sh, sem0, sem1):
    c = lax.axis_index("c")
    s = lax.axis_index("s")
    w = s * NC + c
    pltpu.sync_copy(zeros_hbm, acc_sh.at[pl.ds(s * (NP // NS), NP // NS)])
    plsc.subcore_barrier()

    dummy = zeros_hbm.at[pl.ds(0, 128)]

    def _drain(buf, sem):
        pltpu.make_async_copy(dummy, buf, sem).wait()

    for gbase, nrows in ((0, 40), (40, 39)):
        pltpu.sync_copy(src_hbm.at[w, pl.ds(gbase, nrows)],
                        src_v.at[pl.ds(0, nrows)])
        pltpu.sync_copy(dst_hbm.at[w, pl.ds(gbase, nrows)],
                        dst_v.at[pl.ds(0, nrows)])
        pltpu.async_copy(g_hbm.at[src_v.at[0]], rows0_v, sem0)

        def body(k, carry, nrows=nrows):
            e = k * 2
            _drain(rows0_v, sem0)
            pltpu.async_copy(g_hbm.at[src_v.at[e + 1]], rows1_v, sem1)
            pltpu.sync_copy(rows0_v, acc_sh.at[dst_v.at[e]], add=True)
            _drain(rows1_v, sem1)

            @pl.when(e + 2 < nrows)
            def _():
                pltpu.async_copy(g_hbm.at[src_v.at[e + 2]], rows0_v, sem0)
            pltpu.sync_copy(rows1_v, acc_sh.at[dst_v.at[e + 1]], add=True)
            return carry
        lax.fori_loop(0, nrows // 2, body, 0)
        if nrows % 2:
            _drain(rows0_v, sem0)
            pltpu.sync_copy(rows0_v, acc_sh.at[dst_v.at[nrows - 1]], add=True)

    plsc.subcore_barrier()
    stripe = pl.ds(s * (NP // NS), NP // NS)
    pltpu.sync_copy(acc_sh.at[stripe], part_hbm.at[c, stripe])


@functools.partial(
    pl.kernel, mesh=_mesh, compiler_params=_sc_params,
    out_type=jax.ShapeDtypeStruct((NW, NP), jnp.float32),
    scratch_types=[pltpu.VMEM((ET,), jnp.int32),
                   pltpu.VMEM((ET,), jnp.int32),
                   pltpu.VMEM((NP,), jnp.float32),
                   pltpu.VMEM((NP,), jnp.float32)],
)
def _sc_scalagg(gs_hbm, src_hbm, dst_hbm, sagg_hbm,
                src_v, dst_v, gs_v, acc_v):
    w = lax.axis_index("s") * NC + lax.axis_index("c")
    pltpu.sync_copy(src_hbm.at[w], src_v)
    pltpu.sync_copy(dst_hbm.at[w], dst_v)
    pltpu.sync_copy(gs_hbm, gs_v)

    def zero(i, carry):
        acc_v[pl.ds(i * L, L)] = jnp.zeros((L,), jnp.float32)
        return carry
    lax.fori_loop(0, NVEC, zero, 0)

    def body(i, carry):
        vs = src_v[pl.ds(i * L, L)]
        vd = dst_v[pl.ds(i * L, L)]
        val = plsc.load_gather(gs_v, [vs])
        plsc.addupdate_scatter(acc_v, [vd], val)
        return carry
    lax.fori_loop(0, NVEC, body, 0)

    pltpu.sync_copy(acc_v, sagg_hbm.at[w])


def _tc1_body(x_ref, w_ref, degp_ref, nm_ref, g_ref, st_ref):
    deg = jnp.sum(degp_ref[...], axis=0) + nm_ref[0, 0, :]
    dinv = jnp.where(deg > 0, lax.rsqrt(deg), 0.0)
    h_pre = jnp.dot(x_ref[...], w_ref[...], preferred_element_type=jnp.float32)
    g_ref[...] = h_pre * dinv[:, None]
    st_ref[...] = h_pre * (nm_ref[0, 0, :] * dinv * dinv)[:, None]


def _tc1(x, W, degp, nm3):
    return pl.pallas_call(
        _tc1_body,
        grid=(ROWS,),
        in_specs=[pl.BlockSpec((128, F), lambda i: (i, 0)),
                  pl.BlockSpec((F, F), lambda i: (0, 0)),
                  pl.BlockSpec((NW, 128), lambda i: (0, i)),
                  pl.BlockSpec((1, 1, 128), lambda i: (i, 0, 0))],
        out_specs=[pl.BlockSpec((128, F), lambda i: (i, 0)),
                   pl.BlockSpec((128, F), lambda i: (i, 0))],
        out_shape=[jax.ShapeDtypeStruct((NP, F), jnp.float32),
                   jax.ShapeDtypeStruct((NP, F), jnp.float32)],
    )(x, W, degp, nm3)


def _tc2_body(part_ref, st_ref, degp_ref, nm_ref, b_ref, ws_ref,
              h_ref, spre_ref, gs_ref):
    deg = jnp.sum(degp_ref[...], axis=0) + nm_ref[0, 0, :]
    dinv = jnp.where(deg > 0, lax.rsqrt(deg), 0.0)
    agg = part_ref[0] + part_ref[1]
    h = jnp.maximum(agg * dinv[:, None] + st_ref[...] + b_ref[0, 0, :][None, :], 0.0)
    h_ref[...] = h
    spre = jnp.sum(h * ws_ref[0, 0, :][None, :], axis=1)
    spre_ref[0, 0, :] = spre
    gs_ref[0, 0, :] = spre * dinv


def _tc2(part, st, degp, nm3, b3, ws3):
    return pl.pallas_call(
        _tc2_body,
        grid=(ROWS,),
        in_specs=[pl.BlockSpec((NC, 128, F), lambda i: (0, i, 0)),
                  pl.BlockSpec((128, F), lambda i: (i, 0)),
                  pl.BlockSpec((NW, 128), lambda i: (0, i)),
                  pl.BlockSpec((1, 1, 128), lambda i: (i, 0, 0)),
                  pl.BlockSpec((1, 1, 128), lambda i: (0, 0, 0)),
                  pl.BlockSpec((1, 1, 128), lambda i: (0, 0, 0))],
        out_specs=[pl.BlockSpec((128, F), lambda i: (i, 0)),
                   pl.BlockSpec((1, 1, 128), lambda i: (i, 0, 0)),
                   pl.BlockSpec((1, 1, 128), lambda i: (i, 0, 0))],
        out_shape=[jax.ShapeDtypeStruct((NP, F), jnp.float32),
                   jax.ShapeDtypeStruct((ROWS, 1, 128), jnp.float32),
                   jax.ShapeDtypeStruct((ROWS, 1, 128), jnp.float32)],
    )(part, st, degp, nm3, b3, ws3)


def _tc3_body(sagg_ref, degp_ref, nm_ref, spre_ref, bs_ref, score_ref):
    deg = jnp.sum(degp_ref[...], axis=0) + nm_ref[0, 0, :]
    dinv = jnp.where(deg > 0, lax.rsqrt(deg), 0.0)
    sa = jnp.sum(sagg_ref[...], axis=0)
    score_ref[0, 0, :] = (sa * dinv
                          + spre_ref[0, 0, :] * (nm_ref[0, 0, :] * dinv * dinv)
                          + bs_ref[0, 0, :])


def _tc3(sagg, degp, nm3, spre3, bs3):
    return pl.pallas_call(
        _tc3_body,
        grid=(ROWS,),
        in_specs=[pl.BlockSpec((NW, 128), lambda i: (0, i)),
                  pl.BlockSpec((NW, 128), lambda i: (0, i)),
                  pl.BlockSpec((1, 1, 128), lambda i: (i, 0, 0)),
                  pl.BlockSpec((1, 1, 128), lambda i: (i, 0, 0)),
                  pl.BlockSpec((1, 1, 128), lambda i: (0, 0, 0))],
        out_specs=pl.BlockSpec((1, 1, 128), lambda i: (i, 0, 0)),
        out_shape=jax.ShapeDtypeStruct((ROWS, 1, 128), jnp.float32),
    )(sagg, degp, nm3, spre3, bs3)


def _tc4_body(h_ref, score_ref, keep_ref, batch_ref,
              hp_ref, sums_ref, cnts_ref, maxs_ref):
    i = pl.program_id(0)
    th = jnp.tanh(score_ref[0, 0, :])
    hpb = h_ref[...] * th[:, None]
    hp_ref[...] = hpb
    keep = keep_ref[0, 0, :]
    kb = jnp.where(keep > 0, batch_ref[0, 0, :], G)

    onehot = (kb[:, None] == lax.broadcasted_iota(jnp.int32, (128, G), 1)
              ).astype(jnp.float32)
    psum = lax.dot_general(onehot, hpb, (((0,), (0,)), ((), ())),
                           preferred_element_type=jnp.float32)
    pcnt = jnp.sum(onehot, axis=0)

    neg = jnp.float32(-3.0e38)
    rows = []
    for g in range(G):
        mf = (kb == g).astype(jnp.float32)[:, None]
        rows.append(jnp.max(hpb * mf + neg * (1.0 - mf), axis=0))
    pmax = jnp.stack(rows, axis=0)

    @pl.when(i == 0)
    def _():
        sums_ref[...] = jnp.zeros((G, F), jnp.float32)
        cnts_ref[...] = jnp.zeros((G, 128), jnp.float32)
        maxs_ref[...] = jnp.full((G, F), neg, jnp.float32)

    sums_ref[...] += psum
    cnts_ref[...] += pcnt[:, None]
    maxs_ref[...] = jnp.maximum(maxs_ref[...], pmax)


def _tc4(h, score3, keep3, batch3):
    return pl.pallas_call(
        _tc4_body,
        grid=(ROWS,),
        in_specs=[pl.BlockSpec((128, F), lambda i: (i, 0)),
                  pl.BlockSpec((1, 1, 128), lambda i: (i, 0, 0)),
                  pl.BlockSpec((1, 1, 128), lambda i: (i, 0, 0)),
                  pl.BlockSpec((1, 1, 128), lambda i: (i, 0, 0))],
        out_specs=[pl.BlockSpec((128, F), lambda i: (i, 0)),
                   pl.BlockSpec((G, F), lambda i: (0, 0)),
                   pl.BlockSpec((G, 128), lambda i: (0, 0)),
                   pl.BlockSpec((G, F), lambda i: (0, 0))],
        out_shape=[jax.ShapeDtypeStruct((NP, F), jnp.float32),
                   jax.ShapeDtypeStruct((G, F), jnp.float32),
                   jax.ShapeDtypeStruct((G, 128), jnp.float32),
                   jax.ShapeDtypeStruct((G, F), jnp.float32)],
    )(h, score3, keep3, batch3)


def _tc5_body(xs_ref, wl_ref, bl_ref, out_ref):
    out_ref[...] = jnp.maximum(
        jnp.dot(xs_ref[...], wl_ref[...], preferred_element_type=jnp.float32)
        + bl_ref[0][None, :], 0.0)


def _tc5(xs, Wl, bl2):
    return pl.pallas_call(
        _tc5_body,
        in_specs=[pl.BlockSpec((G, 2 * F), lambda: (0, 0)),
                  pl.BlockSpec((2 * F, F), lambda: (0, 0)),
                  pl.BlockSpec((8, F), lambda: (0, 0))],
        out_specs=pl.BlockSpec((G, F), lambda: (0, 0)),
        out_shape=jax.ShapeDtypeStruct((G, F), jnp.float32),
    )(xs, Wl, bl2)


def _topk_keep(score, valid, batch_pad):
    bmask = jnp.where(valid, batch_pad, G)
    counts = jnp.bincount(bmask, length=G + 1)
    k = jnp.ceil(RATIO * counts).astype(counts.dtype).at[G].set(0)
    order = jnp.lexsort((-score, bmask))
    bsrt = bmask[order]
    starts = jnp.cumsum(counts) - counts
    pos = jnp.arange(NP) - starts[bsrt]
    keep_sorted = pos < k[bsrt]
    return jnp.zeros((NP,), bool).at[order].set(keep_sorted)


def kernel(x, edge_index, batch, W1, b1, Ws1, bs1, W2, b2, Ws2, bs2,
           W3, b3, Ws3, bs3, Wl, bl):
    f32 = jnp.float32
    xp = jnp.zeros((NP, F), f32).at[:N, :].set(x)
    batch_pad = jnp.concatenate([batch.astype(jnp.int32),
                                 jnp.full((NP - N,), G, jnp.int32)])
    batch3 = batch_pad.reshape(ROWS, 1, 128)
    src = jnp.concatenate([edge_index[0].astype(jnp.int32),
                           jnp.zeros((EP - E,), jnp.int32)])
    dst = jnp.concatenate([edge_index[1].astype(jnp.int32),
                           jnp.full((EP - E,), DUMMY, jnp.int32)])
    src2 = src.reshape(NW, ET)
    src3 = src.reshape(NW, ROWS, 128)
    dst2 = dst.reshape(NW, ET)

    zeros_rows = jnp.zeros((NP // NS, F), f32)
    keep_i = jnp.concatenate([jnp.ones((N,), jnp.int32),
                              jnp.zeros((NP - N,), jnp.int32)])
    nm = keep_i.astype(f32)
    dst_cur2 = dst2
    xcur = xp
    xs = None
    for (W, b, Ws, bs) in ((W1, b1, Ws1, bs1), (W2, b2, Ws2, bs2),
                           (W3, b3, Ws3, bs3)):
        nm3 = nm.reshape(ROWS, 1, 128)
        b3r = b.reshape(1, 1, 128)
        ws3r = Ws.reshape(1, 1, 128)
        bs3r = jnp.broadcast_to(bs.reshape(1, 1, 1), (1, 1, 128))

        ndst2, degp = _sc_deg(src2, dst_cur2, keep_i)
        dst_cur2 = ndst2
        dst3 = ndst2.reshape(NW, ROWS, 128)

        g, st = _tc1(xcur, W, degp, nm3)
        part = _sc_rowagg(g, src3, dst3, zeros_rows)
        h, spre3, gs3 = _tc2(part, st, degp, nm3, b3r, ws3r)
        sagg = _sc_scalagg(gs3.reshape(NP), src2, ndst2)
        score3 = _tc3(sagg, degp, nm3, spre3, bs3r)

        score = score3.reshape(NP)
        keep = _topk_keep(score, nm > 0, batch_pad)
        keep_i = keep.astype(jnp.int32)
        keep3 = keep_i.astype(f32).reshape(ROWS, 1, 128)

        hp, sums, cnts, maxs = _tc4(h, score3, keep3, batch3)
        cnt = cnts[:, 0]
        gmp = jnp.where(cnt[:, None] > 0, maxs, 0.0)
        gap = sums / jnp.maximum(cnt, 1.0)[:, None]
        ro = jnp.concatenate([gmp, gap], axis=1)
        xs = ro if xs is None else xs + ro

        nm = keep_i.astype(f32)
        xcur = hp

    bl2 = jnp.broadcast_to(bl[None, :], (8, F))
    return _tc5(xs, Wl, bl2)


# speedup vs baseline: 21.0527x; 1.1621x over previous
import functools
import jax
import jax.numpy as jnp
from jax import lax
from jax.experimental import pallas as pl
from jax.experimental.pallas import tpu as pltpu
from jax.experimental.pallas import tpu_sc as plsc

N = 10000
F = 128
G = 64
RATIO = 0.8
NC, NS, L = 2, 16, 16
NW = NC * NS
NP = 10112
ROWS = NP // 128
DUMMY = N
E = 320000
ET = NP
EP = NW * ET
NVEC = NP // L
CH = 64
NCH = ET // CH

_mesh = plsc.VectorSubcoreMesh(core_axis_name="c", subcore_axis_name="s")
_sc_params = pltpu.CompilerParams(needs_layout_passes=False)


@functools.partial(
    pl.kernel, mesh=_mesh, compiler_params=_sc_params,
    out_type=(jax.ShapeDtypeStruct((NW, ET), jnp.int32),
              jax.ShapeDtypeStruct((NW, NP), jnp.float32)),
    scratch_types=[pltpu.VMEM((ET,), jnp.int32),
                   pltpu.VMEM((ET,), jnp.int32),
                   pltpu.VMEM((NP,), jnp.int32),
                   pltpu.VMEM((ET,), jnp.int32),
                   pltpu.VMEM((NP,), jnp.float32)],
)
def _sc_deg(src_hbm, dst_hbm, keep_hbm, ndst_hbm, degp_hbm,
            src_v, dst_v, keep_v, ndst_v, acc_v):
    w = lax.axis_index("s") * NC + lax.axis_index("c")
    pltpu.sync_copy(src_hbm.at[w], src_v)
    pltpu.sync_copy(dst_hbm.at[w], dst_v)
    pltpu.sync_copy(keep_hbm, keep_v)

    def zero(i, carry):
        acc_v[pl.ds(i * L, L)] = jnp.zeros((L,), jnp.float32)
        return carry
    lax.fori_loop(0, NVEC, zero, 0)

    ones = jnp.ones((L,), jnp.float32)

    def body(i, carry):
        vs = src_v[pl.ds(i * L, L)]
        vd = dst_v[pl.ds(i * L, L)]
        ks = plsc.load_gather(keep_v, [vs])
        kd = plsc.load_gather(keep_v, [vd])
        nd = jnp.where((ks > 0) & (kd > 0), vd, DUMMY)
        ndst_v[pl.ds(i * L, L)] = nd
        plsc.addupdate_scatter(acc_v, [nd], ones)
        return carry
    lax.fori_loop(0, NVEC, body, 0)

    pltpu.sync_copy(ndst_v, ndst_hbm.at[w])
    pltpu.sync_copy(acc_v, degp_hbm.at[w])


@functools.partial(
    pl.kernel, mesh=_mesh, compiler_params=_sc_params,
    out_type=jax.ShapeDtypeStruct((NC, NP, F), jnp.float32),
    scratch_types=[pltpu.VMEM((40, 128), jnp.int32),
                   pltpu.VMEM((40, 128), jnp.int32),
                   pltpu.VMEM((128, F), jnp.float32),
                   pltpu.VMEM((128, F), jnp.float32),
                   pltpu.VMEM_SHARED((NP, F), jnp.float32),
                   pltpu.SemaphoreType.DMA,
                   pltpu.SemaphoreType.DMA],
)
def _sc_rowagg(g_hbm, src_hbm, dst_hbm, zeros_hbm, part_hbm,
               src_v, dst_v, rows0_v, rows1_v, acc_sh, sem0, sem1):
    c = lax.axis_index("c")
    s = lax.axis_index("s")
    w = s * NC + c
    pltpu.sync_copy(zeros_hbm, acc_sh.at[pl.ds(s * (NP // NS), NP // NS)])
    plsc.subcore_barrier()

    dummy = zeros_hbm.at[pl.ds(0, 128)]

    def _drain(buf, sem):
        pltpu.make_async_copy(dummy, buf, sem).wait()

    for gbase, nrows in ((0, 40), (40, 39)):
        pltpu.sync_copy(src_hbm.at[w, pl.ds(gbase, nrows)],
                        src_v.at[pl.ds(0, nrows)])
        pltpu.sync_copy(dst_hbm.at[w, pl.ds(gbase, nrows)],
                        dst_v.at[pl.ds(0, nrows)])
        pltpu.async_copy(g_hbm.at[src_v.at[0]], rows0_v, sem0)

        def body(k, carry, nrows=nrows):
            e = k * 2
            _drain(rows0_v, sem0)
            pltpu.async_copy(g_hbm.at[src_v.at[e + 1]], rows1_v, sem1)
            pltpu.sync_copy(rows0_v, acc_sh.at[dst_v.at[e]], add=True)
            _drain(rows1_v, sem1)

            @pl.when(e + 2 < nrows)
            def _():
                pltpu.async_copy(g_hbm.at[src_v.at[e + 2]], rows0_v, sem0)
            pltpu.sync_copy(rows1_v, acc_sh.at[dst_v.at[e + 1]], add=True)
            return carry
        lax.fori_loop(0, nrows // 2, body, 0)
        if nrows % 2:
            _drain(rows0_v, sem0)
            pltpu.sync_copy(rows0_v, acc_sh.at[dst_v.at[nrows - 1]], add=True)

    plsc.subcore_barrier()
    stripe = pl.ds(s * (NP // NS), NP // NS)
    pltpu.sync_copy(acc_sh.at[stripe], part_hbm.at[c, stripe])


@functools.partial(
    pl.kernel, mesh=_mesh, compiler_params=_sc_params,
    out_type=jax.ShapeDtypeStruct((NW, NP), jnp.float32),
    scratch_types=[pltpu.VMEM((ET,), jnp.int32),
                   pltpu.VMEM((ET,), jnp.int32),
                   pltpu.VMEM((NP,), jnp.float32),
                   pltpu.VMEM((NP,), jnp.float32)],
)
def _sc_scalagg(gs_hbm, src_hbm, dst_hbm, sagg_hbm,
                src_v, dst_v, gs_v, acc_v):
    w = lax.axis_index("s") * NC + lax.axis_index("c")
    pltpu.sync_copy(src_hbm.at[w], src_v)
    pltpu.sync_copy(dst_hbm.at[w], dst_v)
    pltpu.sync_copy(gs_hbm, gs_v)

    def zero(i, carry):
        acc_v[pl.ds(i * L, L)] = jnp.zeros((L,), jnp.float32)
        return carry
    lax.fori_loop(0, NVEC, zero, 0)

    def body(i, carry):
        vs = src_v[pl.ds(i * L, L)]
        vd = dst_v[pl.ds(i * L, L)]
        val = plsc.load_gather(gs_v, [vs])
        plsc.addupdate_scatter(acc_v, [vd], val)
        return carry
    lax.fori_loop(0, NVEC, body, 0)

    pltpu.sync_copy(acc_v, sagg_hbm.at[w])


def _tc1_body(x_ref, w_ref, degp_ref, nm_ref, g_ref, st_ref):
    deg = jnp.sum(degp_ref[...], axis=0) + nm_ref[0, 0, :]
    dinv = jnp.where(deg > 0, lax.rsqrt(deg), 0.0)
    h_pre = jnp.dot(x_ref[...], w_ref[...], preferred_element_type=jnp.float32)
    g_ref[...] = h_pre * dinv[:, None]
    st_ref[...] = h_pre * (nm_ref[0, 0, :] * dinv * dinv)[:, None]


def _tc1(x, W, degp, nm3):
    return pl.pallas_call(
        _tc1_body,
        grid=(ROWS,),
        in_specs=[pl.BlockSpec((128, F), lambda i: (i, 0)),
                  pl.BlockSpec((F, F), lambda i: (0, 0)),
                  pl.BlockSpec((NW, 128), lambda i: (0, i)),
                  pl.BlockSpec((1, 1, 128), lambda i: (i, 0, 0))],
        out_specs=[pl.BlockSpec((128, F), lambda i: (i, 0)),
                   pl.BlockSpec((128, F), lambda i: (i, 0))],
        out_shape=[jax.ShapeDtypeStruct((NP, F), jnp.float32),
                   jax.ShapeDtypeStruct((NP, F), jnp.float32)],
    )(x, W, degp, nm3)


def _tc2_body(part_ref, st_ref, degp_ref, nm_ref, b_ref, ws_ref,
              h_ref, spre_ref, gs_ref):
    deg = jnp.sum(degp_ref[...], axis=0) + nm_ref[0, 0, :]
    dinv = jnp.where(deg > 0, lax.rsqrt(deg), 0.0)
    agg = part_ref[0] + part_ref[1]
    h = jnp.maximum(agg * dinv[:, None] + st_ref[...] + b_ref[0, 0, :][None, :], 0.0)
    h_ref[...] = h
    spre = jnp.sum(h * ws_ref[0, 0, :][None, :], axis=1)
    spre_ref[0, 0, :] = spre
    gs_ref[0, 0, :] = spre * dinv


def _tc2(part, st, degp, nm3, b3, ws3):
    return pl.pallas_call(
        _tc2_body,
        grid=(ROWS,),
        in_specs=[pl.BlockSpec((NC, 128, F), lambda i: (0, i, 0)),
                  pl.BlockSpec((128, F), lambda i: (i, 0)),
                  pl.BlockSpec((NW, 128), lambda i: (0, i)),
                  pl.BlockSpec((1, 1, 128), lambda i: (i, 0, 0)),
                  pl.BlockSpec((1, 1, 128), lambda i: (0, 0, 0)),
                  pl.BlockSpec((1, 1, 128), lambda i: (0, 0, 0))],
        out_specs=[pl.BlockSpec((128, F), lambda i: (i, 0)),
                   pl.BlockSpec((1, 1, 128), lambda i: (i, 0, 0)),
                   pl.BlockSpec((1, 1, 128), lambda i: (i, 0, 0))],
        out_shape=[jax.ShapeDtypeStruct((NP, F), jnp.float32),
                   jax.ShapeDtypeStruct((ROWS, 1, 128), jnp.float32),
                   jax.ShapeDtypeStruct((ROWS, 1, 128), jnp.float32)],
    )(part, st, degp, nm3, b3, ws3)


def _tc3_body(sagg_ref, degp_ref, nm_ref, spre_ref, bs_ref, score_ref):
    deg = jnp.sum(degp_ref[...], axis=0) + nm_ref[0, 0, :]
    dinv = jnp.where(deg > 0, lax.rsqrt(deg), 0.0)
    sa = jnp.sum(sagg_ref[...], axis=0)
    score_ref[0, 0, :] = (sa * dinv
                          + spre_ref[0, 0, :] * (nm_ref[0, 0, :] * dinv * dinv)
                          + bs_ref[0, 0, :])


def _tc3(sagg, degp, nm3, spre3, bs3):
    return pl.pallas_call(
        _tc3_body,
        grid=(ROWS,),
        in_specs=[pl.BlockSpec((NW, 128), lambda i: (0, i)),
                  pl.BlockSpec((NW, 128), lambda i: (0, i)),
                  pl.BlockSpec((1, 1, 128), lambda i: (i, 0, 0)),
                  pl.BlockSpec((1, 1, 128), lambda i: (i, 0, 0)),
                  pl.BlockSpec((1, 1, 128), lambda i: (0, 0, 0))],
        out_specs=pl.BlockSpec((1, 1, 128), lambda i: (i, 0, 0)),
        out_shape=jax.ShapeDtypeStruct((ROWS, 1, 128), jnp.float32),
    )(sagg, degp, nm3, spre3, bs3)


def _tc4_body(h_ref, score_ref, keep_ref, batch_ref,
              hp_ref, sums_ref, cnts_ref, maxs_ref):
    i = pl.program_id(0)
    th = jnp.tanh(score_ref[0, 0, :])
    hpb = h_ref[...] * th[:, None]
    hp_ref[...] = hpb
    keep = keep_ref[0, 0, :]
    kb = jnp.where(keep > 0, batch_ref[0, 0, :], G)

    onehot = (kb[:, None] == lax.broadcasted_iota(jnp.int32, (128, G), 1)
              ).astype(jnp.float32)
    psum = lax.dot_general(onehot, hpb, (((0,), (0,)), ((), ())),
                           preferred_element_type=jnp.float32)
    pcnt = jnp.sum(onehot, axis=0)

    neg = jnp.float32(-3.0e38)
    rows = []
    for g in range(G):
        mf = (kb == g).astype(jnp.float32)[:, None]
        rows.append(jnp.max(hpb * mf + neg * (1.0 - mf), axis=0))
    pmax = jnp.stack(rows, axis=0)

    @pl.when(i == 0)
    def _():
        sums_ref[...] = jnp.zeros((G, F), jnp.float32)
        cnts_ref[...] = jnp.zeros((G, 128), jnp.float32)
        maxs_ref[...] = jnp.full((G, F), neg, jnp.float32)

    sums_ref[...] += psum
    cnts_ref[...] += pcnt[:, None]
    maxs_ref[...] = jnp.maximum(maxs_ref[...], pmax)


def _tc4(h, score3, keep3, batch3):
    return pl.pallas_call(
        _tc4_body,
        grid=(ROWS,),
        in_specs=[pl.BlockSpec((128, F), lambda i: (i, 0)),
                  pl.BlockSpec((1, 1, 128), lambda i: (i, 0, 0)),
                  pl.BlockSpec((1, 1, 128), lambda i: (i, 0, 0)),
                  pl.BlockSpec((1, 1, 128), lambda i: (i, 0, 0))],
        out_specs=[pl.BlockSpec((128, F), lambda i: (i, 0)),
                   pl.BlockSpec((G, F), lambda i: (0, 0)),
                   pl.BlockSpec((G, 128), lambda i: (0, 0)),
                   pl.BlockSpec((G, F), lambda i: (0, 0))],
        out_shape=[jax.ShapeDtypeStruct((NP, F), jnp.float32),
                   jax.ShapeDtypeStruct((G, F), jnp.float32),
                   jax.ShapeDtypeStruct((G, 128), jnp.float32),
                   jax.ShapeDtypeStruct((G, F), jnp.float32)],
    )(h, score3, keep3, batch3)


def _tc5_body(xs_ref, wl_ref, bl_ref, out_ref):
    out_ref[...] = jnp.maximum(
        jnp.dot(xs_ref[...], wl_ref[...], preferred_element_type=jnp.float32)
        + bl_ref[0][None, :], 0.0)


def _tc5(xs, Wl, bl2):
    return pl.pallas_call(
        _tc5_body,
        in_specs=[pl.BlockSpec((G, 2 * F), lambda: (0, 0)),
                  pl.BlockSpec((2 * F, F), lambda: (0, 0)),
                  pl.BlockSpec((8, F), lambda: (0, 0))],
        out_specs=pl.BlockSpec((G, F), lambda: (0, 0)),
        out_shape=jax.ShapeDtypeStruct((G, F), jnp.float32),
    )(xs, Wl, bl2)


def _tcrank_body(sa_ref, ba_ref, va_ref, st_ref, bt_ref, lo_ref, hi_ref,
                 rank_ref):
    i = pl.program_id(0)
    rs = st_ref[0, 0, :][None, :]
    rb = bt_ref[0, 0, :][None, :]
    ridx = lax.broadcasted_iota(jnp.int32, (1, 128), 1) + i * 128
    lo = lo_ref[0, 0, 0]
    hi = hi_ref[0, 0, 0]

    def body(c, acc):
        cs = sa_ref[c, 0, :][:, None]
        cb = ba_ref[c, 0, :][:, None]
        cv = va_ref[c, 0, :][:, None]
        cidx = lax.broadcasted_iota(jnp.int32, (128, 1), 0) + c * 128
        beats = (cs > rs) | ((cs == rs) & (cidx < ridx))
        contrib = jnp.where((cb == rb) & (cv > 0) & beats, 1.0, 0.0)
        return acc + jnp.sum(contrib, axis=0)
    rank = lax.fori_loop(lo, hi + 1, body, jnp.zeros((128,), jnp.float32))
    rank_ref[0, 0, :] = rank


def _tc_rank(score3, batch3f, valid3, lo3, hi3):
    whole = pl.BlockSpec((ROWS, 1, 128), lambda i: (0, 0, 0))
    tile = pl.BlockSpec((1, 1, 128), lambda i: (i, 0, 0))
    return pl.pallas_call(
        _tcrank_body,
        grid=(ROWS,),
        in_specs=[whole, whole, whole, tile, tile, tile, tile],
        out_specs=tile,
        out_shape=jax.ShapeDtypeStruct((ROWS, 1, 128), jnp.float32),
    )(score3, batch3f, valid3, score3, batch3f, lo3, hi3)


def _topk_keep_pallas(score3, nm, batch_pad, batch3f, tile_min, tile_max):
    valid = nm > 0
    bmask = jnp.where(valid, batch_pad, G)
    counts = jnp.bincount(bmask, length=G + 1)
    k = jnp.ceil(RATIO * counts).astype(jnp.int32).at[G].set(0)
    lo_t = jnp.searchsorted(tile_max, tile_min, side='left').astype(jnp.int32)
    hi_t = (jnp.searchsorted(tile_min, tile_max, side='right') - 1).astype(jnp.int32)
    lo3 = jnp.broadcast_to(lo_t[:, None, None], (ROWS, 1, 128))
    hi3 = jnp.broadcast_to(hi_t[:, None, None], (ROWS, 1, 128))
    valid3 = nm.reshape(ROWS, 1, 128)
    rank = _tc_rank(score3, batch3f, valid3, lo3, hi3).reshape(NP)
    return valid & (rank < k[bmask].astype(jnp.float32))


def _topk_keep(score, valid, batch_pad):
    bmask = jnp.where(valid, batch_pad, G)
    counts = jnp.bincount(bmask, length=G + 1)
    k = jnp.ceil(RATIO * counts).astype(counts.dtype).at[G].set(0)
    order = jnp.lexsort((-score, bmask))
    bsrt = bmask[order]
    starts = jnp.cumsum(counts) - counts
    pos = jnp.arange(NP) - starts[bsrt]
    keep_sorted = pos < k[bsrt]
    return jnp.zeros((NP,), bool).at[order].set(keep_sorted)


def kernel(x, edge_index, batch, W1, b1, Ws1, bs1, W2, b2, Ws2, bs2,
           W3, b3, Ws3, bs3, Wl, bl):
    f32 = jnp.float32
    xp = jnp.zeros((NP, F), f32).at[:N, :].set(x)
    batch_pad = jnp.concatenate([batch.astype(jnp.int32),
                                 jnp.full((NP - N,), G, jnp.int32)])
    batch3 = batch_pad.reshape(ROWS, 1, 128)
    batch3f = batch_pad.astype(f32).reshape(ROWS, 1, 128)
    bt2 = batch_pad.reshape(ROWS, 128)
    tile_min = bt2.min(axis=1).astype(jnp.int32)
    tile_max = bt2.max(axis=1).astype(jnp.int32)
    src = jnp.concatenate([edge_index[0].astype(jnp.int32),
                           jnp.zeros((EP - E,), jnp.int32)])
    dst = jnp.concatenate([edge_index[1].astype(jnp.int32),
                           jnp.full((EP - E,), DUMMY, jnp.int32)])
    src2 = src.reshape(NW, ET)
    src3 = src.reshape(NW, ROWS, 128)
    dst2 = dst.reshape(NW, ET)

    zeros_rows = jnp.zeros((NP // NS, F), f32)
    keep_i = jnp.concatenate([jnp.ones((N,), jnp.int32),
                              jnp.zeros((NP - N,), jnp.int32)])
    nm = keep_i.astype(f32)
    dst_cur2 = dst2
    xcur = xp
    xs = None
    for (W, b, Ws, bs) in ((W1, b1, Ws1, bs1), (W2, b2, Ws2, bs2),
                           (W3, b3, Ws3, bs3)):
        nm3 = nm.reshape(ROWS, 1, 128)
        b3r = b.reshape(1, 1, 128)
        ws3r = Ws.reshape(1, 1, 128)
        bs3r = jnp.broadcast_to(bs.reshape(1, 1, 1), (1, 1, 128))

        ndst2, degp = _sc_deg(src2, dst_cur2, keep_i)
        dst_cur2 = ndst2
        dst3 = ndst2.reshape(NW, ROWS, 128)

        g, st = _tc1(xcur, W, degp, nm3)
        part = _sc_rowagg(g, src3, dst3, zeros_rows)
        h, spre3, gs3 = _tc2(part, st, degp, nm3, b3r, ws3r)
        sagg = _sc_scalagg(gs3.reshape(NP), src2, ndst2)
        score3 = _tc3(sagg, degp, nm3, spre3, bs3r)

        keep = _topk_keep_pallas(score3, nm, batch_pad, batch3f,
                                 tile_min, tile_max)
        keep_i = keep.astype(jnp.int32)
        keep3 = keep_i.astype(f32).reshape(ROWS, 1, 128)

        hp, sums, cnts, maxs = _tc4(h, score3, keep3, batch3)
        cnt = cnts[:, 0]
        gmp = jnp.where(cnt[:, None] > 0, maxs, 0.0)
        gap = sums / jnp.maximum(cnt, 1.0)[:, None]
        ro = jnp.concatenate([gmp, gap], axis=1)
        xs = ro if xs is None else xs + ro

        nm = keep_i.astype(f32)
        xcur = hp

    bl2 = jnp.broadcast_to(bl[None, :], (8, F))
    return _tc5(xs, Wl, bl2)


# speedup vs baseline: 22.5902x; 1.0730x over previous
import functools
import jax
import jax.numpy as jnp
from jax import lax
from jax.experimental import pallas as pl
from jax.experimental.pallas import tpu as pltpu
from jax.experimental.pallas import tpu_sc as plsc

N = 10000
F = 128
G = 64
RATIO = 0.8
NC, NS, L = 2, 16, 16
NW = NC * NS
NP = 10112
ROWS = NP // 128
DUMMY = N
E = 320000
ET = NP
EP = NW * ET
NVEC = NP // L
CH = 64
NCH = ET // CH

_mesh = plsc.VectorSubcoreMesh(core_axis_name="c", subcore_axis_name="s")
_sc_params = pltpu.CompilerParams(needs_layout_passes=False)


@functools.partial(
    pl.kernel, mesh=_mesh, compiler_params=_sc_params,
    out_type=(jax.ShapeDtypeStruct((NW, ET), jnp.int32),
              jax.ShapeDtypeStruct((NW, NP), jnp.float32)),
    scratch_types=[pltpu.VMEM((ET,), jnp.int32),
                   pltpu.VMEM((ET,), jnp.int32),
                   pltpu.VMEM((NP,), jnp.int32),
                   pltpu.VMEM((ET,), jnp.int32),
                   pltpu.VMEM((NP,), jnp.float32)],
)
def _sc_deg(src_hbm, dst_hbm, keep_hbm, ndst_hbm, degp_hbm,
            src_v, dst_v, keep_v, ndst_v, acc_v):
    w = lax.axis_index("s") * NC + lax.axis_index("c")
    pltpu.sync_copy(src_hbm.at[w], src_v)
    pltpu.sync_copy(dst_hbm.at[w], dst_v)
    pltpu.sync_copy(keep_hbm, keep_v)

    def zero(i, carry):
        acc_v[pl.ds(i * L, L)] = jnp.zeros((L,), jnp.float32)
        return carry
    lax.fori_loop(0, NVEC, zero, 0)

    ones = jnp.ones((L,), jnp.float32)

    def body(i, carry):
        vs = src_v[pl.ds(i * L, L)]
        vd = dst_v[pl.ds(i * L, L)]
        ks = plsc.load_gather(keep_v, [vs])
        kd = plsc.load_gather(keep_v, [vd])
        nd = jnp.where((ks > 0) & (kd > 0), vd, DUMMY)
        ndst_v[pl.ds(i * L, L)] = nd
        plsc.addupdate_scatter(acc_v, [nd], ones)
        return carry
    lax.fori_loop(0, NVEC, body, 0)

    pltpu.sync_copy(ndst_v, ndst_hbm.at[w])
    pltpu.sync_copy(acc_v, degp_hbm.at[w])


@functools.partial(
    pl.kernel, mesh=_mesh, compiler_params=_sc_params,
    out_type=jax.ShapeDtypeStruct((NC, NP, F), jnp.float32),
    scratch_types=[pltpu.VMEM((40, 128), jnp.int32),
                   pltpu.VMEM((40, 128), jnp.int32),
                   pltpu.VMEM((128, F), jnp.float32),
                   pltpu.VMEM((128, F), jnp.float32),
                   pltpu.VMEM_SHARED((NP, F), jnp.float32),
                   pltpu.SemaphoreType.DMA,
                   pltpu.SemaphoreType.DMA],
)
def _sc_rowagg(g_hbm, src_hbm, dst_hbm, zeros_hbm, part_hbm,
               src_v, dst_v, rows0_v, rows1_v, acc_sh, sem0, sem1):
    c = lax.axis_index("c")
    s = lax.axis_index("s")
    w = s * NC + c
    pltpu.sync_copy(zeros_hbm, acc_sh.at[pl.ds(s * (NP // NS), NP // NS)])
    plsc.subcore_barrier()

    dummy = zeros_hbm.at[pl.ds(0, 128)]

    def _drain(buf, sem):
        pltpu.make_async_copy(dummy, buf, sem).wait()

    for gbase, nrows in ((0, 40), (40, 39)):
        pltpu.sync_copy(src_hbm.at[w, pl.ds(gbase, nrows)],
                        src_v.at[pl.ds(0, nrows)])
        pltpu.sync_copy(dst_hbm.at[w, pl.ds(gbase, nrows)],
                        dst_v.at[pl.ds(0, nrows)])
        pltpu.async_copy(g_hbm.at[src_v.at[0]], rows0_v, sem0)

        def body(k, carry, nrows=nrows):
            e = k * 2
            _drain(rows0_v, sem0)
            pltpu.async_copy(g_hbm.at[src_v.at[e + 1]], rows1_v, sem1)
            pltpu.sync_copy(rows0_v, acc_sh.at[dst_v.at[e]], add=True)
            _drain(rows1_v, sem1)

            @pl.when(e + 2 < nrows)
            def _():
                pltpu.async_copy(g_hbm.at[src_v.at[e + 2]], rows0_v, sem0)
            pltpu.sync_copy(rows1_v, acc_sh.at[dst_v.at[e + 1]], add=True)
            return carry
        lax.fori_loop(0, nrows // 2, body, 0)
        if nrows % 2:
            _drain(rows0_v, sem0)
            pltpu.sync_copy(rows0_v, acc_sh.at[dst_v.at[nrows - 1]], add=True)

    plsc.subcore_barrier()
    stripe = pl.ds(s * (NP // NS), NP // NS)
    pltpu.sync_copy(acc_sh.at[stripe], part_hbm.at[c, stripe])


@functools.partial(
    pl.kernel, mesh=_mesh, compiler_params=_sc_params,
    out_type=jax.ShapeDtypeStruct((NW, NP), jnp.float32),
    scratch_types=[pltpu.VMEM((ET,), jnp.int32),
                   pltpu.VMEM((ET,), jnp.int32),
                   pltpu.VMEM((NP,), jnp.float32),
                   pltpu.VMEM((NP,), jnp.float32)],
)
def _sc_scalagg(gs_hbm, src_hbm, dst_hbm, sagg_hbm,
                src_v, dst_v, gs_v, acc_v):
    w = lax.axis_index("s") * NC + lax.axis_index("c")
    pltpu.sync_copy(src_hbm.at[w], src_v)
    pltpu.sync_copy(dst_hbm.at[w], dst_v)
    pltpu.sync_copy(gs_hbm, gs_v)

    def zero(i, carry):
        acc_v[pl.ds(i * L, L)] = jnp.zeros((L,), jnp.float32)
        return carry
    lax.fori_loop(0, NVEC, zero, 0)

    def body(i, carry):
        vs = src_v[pl.ds(i * L, L)]
        vd = dst_v[pl.ds(i * L, L)]
        val = plsc.load_gather(gs_v, [vs])
        plsc.addupdate_scatter(acc_v, [vd], val)
        return carry
    lax.fori_loop(0, NVEC, body, 0)

    pltpu.sync_copy(acc_v, sagg_hbm.at[w])


def _tc1_body(x_ref, w_ref, degp_ref, nm_ref, g_ref, st_ref):
    deg = jnp.sum(degp_ref[...], axis=0) + nm_ref[0, 0, :]
    dinv = jnp.where(deg > 0, lax.rsqrt(deg), 0.0)
    h_pre = jnp.dot(x_ref[...], w_ref[...], preferred_element_type=jnp.float32)
    g_ref[...] = h_pre * dinv[:, None]
    st_ref[...] = h_pre * (nm_ref[0, 0, :] * dinv * dinv)[:, None]


def _tc1(x, W, degp, nm3):
    return pl.pallas_call(
        _tc1_body,
        grid=(ROWS,),
        in_specs=[pl.BlockSpec((128, F), lambda i: (i, 0)),
                  pl.BlockSpec((F, F), lambda i: (0, 0)),
                  pl.BlockSpec((NW, 128), lambda i: (0, i)),
                  pl.BlockSpec((1, 1, 128), lambda i: (i, 0, 0))],
        out_specs=[pl.BlockSpec((128, F), lambda i: (i, 0)),
                   pl.BlockSpec((128, F), lambda i: (i, 0))],
        out_shape=[jax.ShapeDtypeStruct((NP, F), jnp.float32),
                   jax.ShapeDtypeStruct((NP, F), jnp.float32)],
    )(x, W, degp, nm3)


def _tc2_body(part_ref, st_ref, degp_ref, nm_ref, b_ref, ws_ref,
              h_ref, spre_ref, gs_ref):
    deg = jnp.sum(degp_ref[...], axis=0) + nm_ref[0, 0, :]
    dinv = jnp.where(deg > 0, lax.rsqrt(deg), 0.0)
    agg = part_ref[0] + part_ref[1]
    h = jnp.maximum(agg * dinv[:, None] + st_ref[...] + b_ref[0, 0, :][None, :], 0.0)
    h_ref[...] = h
    spre = jnp.sum(h * ws_ref[0, 0, :][None, :], axis=1)
    spre_ref[0, 0, :] = spre
    gs_ref[0, 0, :] = spre * dinv


def _tc2(part, st, degp, nm3, b3, ws3):
    return pl.pallas_call(
        _tc2_body,
        grid=(ROWS,),
        in_specs=[pl.BlockSpec((NC, 128, F), lambda i: (0, i, 0)),
                  pl.BlockSpec((128, F), lambda i: (i, 0)),
                  pl.BlockSpec((NW, 128), lambda i: (0, i)),
                  pl.BlockSpec((1, 1, 128), lambda i: (i, 0, 0)),
                  pl.BlockSpec((1, 1, 128), lambda i: (0, 0, 0)),
                  pl.BlockSpec((1, 1, 128), lambda i: (0, 0, 0))],
        out_specs=[pl.BlockSpec((128, F), lambda i: (i, 0)),
                   pl.BlockSpec((1, 1, 128), lambda i: (i, 0, 0)),
                   pl.BlockSpec((1, 1, 128), lambda i: (i, 0, 0))],
        out_shape=[jax.ShapeDtypeStruct((NP, F), jnp.float32),
                   jax.ShapeDtypeStruct((ROWS, 1, 128), jnp.float32),
                   jax.ShapeDtypeStruct((ROWS, 1, 128), jnp.float32)],
    )(part, st, degp, nm3, b3, ws3)


def _tc3_body(sagg_ref, degp_ref, nm_ref, spre_ref, bs_ref, score_ref):
    deg = jnp.sum(degp_ref[...], axis=0) + nm_ref[0, 0, :]
    dinv = jnp.where(deg > 0, lax.rsqrt(deg), 0.0)
    sa = jnp.sum(sagg_ref[...], axis=0)
    score_ref[0, 0, :] = (sa * dinv
                          + spre_ref[0, 0, :] * (nm_ref[0, 0, :] * dinv * dinv)
                          + bs_ref[0, 0, :])


def _tc3(sagg, degp, nm3, spre3, bs3):
    return pl.pallas_call(
        _tc3_body,
        grid=(ROWS,),
        in_specs=[pl.BlockSpec((NW, 128), lambda i: (0, i)),
                  pl.BlockSpec((NW, 128), lambda i: (0, i)),
                  pl.BlockSpec((1, 1, 128), lambda i: (i, 0, 0)),
                  pl.BlockSpec((1, 1, 128), lambda i: (i, 0, 0)),
                  pl.BlockSpec((1, 1, 128), lambda i: (0, 0, 0))],
        out_specs=pl.BlockSpec((1, 1, 128), lambda i: (i, 0, 0)),
        out_shape=jax.ShapeDtypeStruct((ROWS, 1, 128), jnp.float32),
    )(sagg, degp, nm3, spre3, bs3)


def _tc4_body(h_ref, score_ref, keep_ref, batch_ref, lo_ref, hi_ref,
              hp_ref, sums_ref, cnts_ref, maxs_ref):
    i = pl.program_id(0)
    th = jnp.tanh(score_ref[0, 0, :])
    hpb = h_ref[...] * th[:, None]
    hp_ref[...] = hpb
    keep = keep_ref[0, 0, :]
    kb = jnp.where(keep > 0, batch_ref[0, 0, :], G)

    onehot = (kb[:, None] == lax.broadcasted_iota(jnp.int32, (128, G), 1)
              ).astype(jnp.float32)
    psum = lax.dot_general(onehot, hpb, (((0,), (0,)), ((), ())),
                           preferred_element_type=jnp.float32)
    pcnt = jnp.sum(onehot, axis=0)

    neg = jnp.float32(-3.0e38)

    @pl.when(i == 0)
    def _():
        sums_ref[...] = jnp.zeros((G, F), jnp.float32)
        cnts_ref[...] = jnp.zeros((G, 128), jnp.float32)
        maxs_ref[...] = jnp.full((G, F), neg, jnp.float32)

    sums_ref[...] += psum
    cnts_ref[...] += pcnt[:, None]

    lo = lo_ref[0, 0, 0]
    hi = jnp.minimum(hi_ref[0, 0, 0], G - 1)

    def body(g, carry):
        mf = (kb == g).astype(jnp.float32)[:, None]
        mg = jnp.max(hpb * mf + neg * (1.0 - mf), axis=0)
        maxs_ref[pl.ds(g, 1), :] = jnp.maximum(maxs_ref[pl.ds(g, 1), :],
                                               mg[None, :])
        return carry
    lax.fori_loop(lo, hi + 1, body, 0)


def _tc4(h, score3, keep3, batch3, lo3b, hi3b):
    return pl.pallas_call(
        _tc4_body,
        grid=(ROWS,),
        in_specs=[pl.BlockSpec((128, F), lambda i: (i, 0)),
                  pl.BlockSpec((1, 1, 128), lambda i: (i, 0, 0)),
                  pl.BlockSpec((1, 1, 128), lambda i: (i, 0, 0)),
                  pl.BlockSpec((1, 1, 128), lambda i: (i, 0, 0)),
                  pl.BlockSpec((1, 1, 128), lambda i: (i, 0, 0)),
                  pl.BlockSpec((1, 1, 128), lambda i: (i, 0, 0))],
        out_specs=[pl.BlockSpec((128, F), lambda i: (i, 0)),
                   pl.BlockSpec((G, F), lambda i: (0, 0)),
                   pl.BlockSpec((G, 128), lambda i: (0, 0)),
                   pl.BlockSpec((G, F), lambda i: (0, 0))],
        out_shape=[jax.ShapeDtypeStruct((NP, F), jnp.float32),
                   jax.ShapeDtypeStruct((G, F), jnp.float32),
                   jax.ShapeDtypeStruct((G, 128), jnp.float32),
                   jax.ShapeDtypeStruct((G, F), jnp.float32)],
    )(h, score3, keep3, batch3, lo3b, hi3b)


def _tc5_body(xs_ref, wl_ref, bl_ref, out_ref):
    out_ref[...] = jnp.maximum(
        jnp.dot(xs_ref[...], wl_ref[...], preferred_element_type=jnp.float32)
        + bl_ref[0][None, :], 0.0)


def _tc5(xs, Wl, bl2):
    return pl.pallas_call(
        _tc5_body,
        in_specs=[pl.BlockSpec((G, 2 * F), lambda: (0, 0)),
                  pl.BlockSpec((2 * F, F), lambda: (0, 0)),
                  pl.BlockSpec((8, F), lambda: (0, 0))],
        out_specs=pl.BlockSpec((G, F), lambda: (0, 0)),
        out_shape=jax.ShapeDtypeStruct((G, F), jnp.float32),
    )(xs, Wl, bl2)


def _tcrank_body(sa_ref, ba_ref, va_ref, st_ref, bt_ref, lo_ref, hi_ref,
                 rank_ref):
    i = pl.program_id(0)
    rs = st_ref[0, 0, :][None, :]
    rb = bt_ref[0, 0, :][None, :]
    ridx = lax.broadcasted_iota(jnp.int32, (1, 128), 1) + i * 128
    lo = lo_ref[0, 0, 0]
    hi = hi_ref[0, 0, 0]

    def body(c, acc):
        cs = sa_ref[c, 0, :][:, None]
        cb = ba_ref[c, 0, :][:, None]
        cv = va_ref[c, 0, :][:, None]
        cidx = lax.broadcasted_iota(jnp.int32, (128, 1), 0) + c * 128
        beats = (cs > rs) | ((cs == rs) & (cidx < ridx))
        contrib = jnp.where((cb == rb) & (cv > 0) & beats, 1.0, 0.0)
        return acc + jnp.sum(contrib, axis=0)
    rank = lax.fori_loop(lo, hi + 1, body, jnp.zeros((128,), jnp.float32))
    rank_ref[0, 0, :] = rank


def _tc_rank(score3, batch3f, valid3, lo3, hi3):
    whole = pl.BlockSpec((ROWS, 1, 128), lambda i: (0, 0, 0))
    tile = pl.BlockSpec((1, 1, 128), lambda i: (i, 0, 0))
    return pl.pallas_call(
        _tcrank_body,
        grid=(ROWS,),
        in_specs=[whole, whole, whole, tile, tile, tile, tile],
        out_specs=tile,
        out_shape=jax.ShapeDtypeStruct((ROWS, 1, 128), jnp.float32),
    )(score3, batch3f, valid3, score3, batch3f, lo3, hi3)


def _topk_keep_pallas(score3, nm, batch_pad, batch3f, tile_min, tile_max):
    valid = nm > 0
    bmask = jnp.where(valid, batch_pad, G)
    counts = jnp.bincount(bmask, length=G + 1)
    k = jnp.ceil(RATIO * counts).astype(jnp.int32).at[G].set(0)
    lo_t = jnp.searchsorted(tile_max, tile_min, side='left').astype(jnp.int32)
    hi_t = (jnp.searchsorted(tile_min, tile_max, side='right') - 1).astype(jnp.int32)
    lo3 = jnp.broadcast_to(lo_t[:, None, None], (ROWS, 1, 128))
    hi3 = jnp.broadcast_to(hi_t[:, None, None], (ROWS, 1, 128))
    valid3 = nm.reshape(ROWS, 1, 128)
    rank = _tc_rank(score3, batch3f, valid3, lo3, hi3).reshape(NP)
    return valid & (rank < k[bmask].astype(jnp.float32))


def _topk_keep(score, valid, batch_pad):
    bmask = jnp.where(valid, batch_pad, G)
    counts = jnp.bincount(bmask, length=G + 1)
    k = jnp.ceil(RATIO * counts).astype(counts.dtype).at[G].set(0)
    order = jnp.lexsort((-score, bmask))
    bsrt = bmask[order]
    starts = jnp.cumsum(counts) - counts
    pos = jnp.arange(NP) - starts[bsrt]
    keep_sorted = pos < k[bsrt]
    return jnp.zeros((NP,), bool).at[order].set(keep_sorted)


def kernel(x, edge_index, batch, W1, b1, Ws1, bs1, W2, b2, Ws2, bs2,
           W3, b3, Ws3, bs3, Wl, bl):
    f32 = jnp.float32
    xp = jnp.zeros((NP, F), f32).at[:N, :].set(x)
    batch_pad = jnp.concatenate([batch.astype(jnp.int32),
                                 jnp.full((NP - N,), G, jnp.int32)])
    batch3 = batch_pad.reshape(ROWS, 1, 128)
    batch3f = batch_pad.astype(f32).reshape(ROWS, 1, 128)
    bt2 = batch_pad.reshape(ROWS, 128)
    tile_min = bt2.min(axis=1).astype(jnp.int32)
    tile_max = bt2.max(axis=1).astype(jnp.int32)
    lo3b = jnp.broadcast_to(tile_min[:, None, None], (ROWS, 1, 128))
    hi3b = jnp.broadcast_to(tile_max[:, None, None], (ROWS, 1, 128))
    src = jnp.concatenate([edge_index[0].astype(jnp.int32),
                           jnp.zeros((EP - E,), jnp.int32)])
    dst = jnp.concatenate([edge_index[1].astype(jnp.int32),
                           jnp.full((EP - E,), DUMMY, jnp.int32)])
    src2 = src.reshape(NW, ET)
    src3 = src.reshape(NW, ROWS, 128)
    dst2 = dst.reshape(NW, ET)

    zeros_rows = jnp.zeros((NP // NS, F), f32)
    keep_i = jnp.concatenate([jnp.ones((N,), jnp.int32),
                              jnp.zeros((NP - N,), jnp.int32)])
    nm = keep_i.astype(f32)
    dst_cur2 = dst2
    xcur = xp
    xs = None
    for (W, b, Ws, bs) in ((W1, b1, Ws1, bs1), (W2, b2, Ws2, bs2),
                           (W3, b3, Ws3, bs3)):
        nm3 = nm.reshape(ROWS, 1, 128)
        b3r = b.reshape(1, 1, 128)
        ws3r = Ws.reshape(1, 1, 128)
        bs3r = jnp.broadcast_to(bs.reshape(1, 1, 1), (1, 1, 128))

        ndst2, degp = _sc_deg(src2, dst_cur2, keep_i)
        dst_cur2 = ndst2
        dst3 = ndst2.reshape(NW, ROWS, 128)

        g, st = _tc1(xcur, W, degp, nm3)
        part = _sc_rowagg(g, src3, dst3, zeros_rows)
        h, spre3, gs3 = _tc2(part, st, degp, nm3, b3r, ws3r)
        sagg = _sc_scalagg(gs3.reshape(NP), src2, ndst2)
        score3 = _tc3(sagg, degp, nm3, spre3, bs3r)

        keep = _topk_keep_pallas(score3, nm, batch_pad, batch3f,
                                 tile_min, tile_max)
        keep_i = keep.astype(jnp.int32)
        keep3 = keep_i.astype(f32).reshape(ROWS, 1, 128)

        hp, sums, cnts, maxs = _tc4(h, score3, keep3, batch3, lo3b, hi3b)
        cnt = cnts[:, 0]
        gmp = jnp.where(cnt[:, None] > 0, maxs, 0.0)
        gap = sums / jnp.maximum(cnt, 1.0)[:, None]
        ro = jnp.concatenate([gmp, gap], axis=1)
        xs = ro if xs is None else xs + ro

        nm = keep_i.astype(f32)
        xcur = hp

    bl2 = jnp.broadcast_to(bl[None, :], (8, F))
    return _tc5(xs, Wl, bl2)


# speedup vs baseline: 22.6245x; 1.0015x over previous
import functools
import jax
import jax.numpy as jnp
from jax import lax
from jax.experimental import pallas as pl
from jax.experimental.pallas import tpu as pltpu
from jax.experimental.pallas import tpu_sc as plsc

N = 10000
F = 128
G = 64
RATIO = 0.8
NC, NS, L = 2, 16, 16
NW = NC * NS
NP = 10112
ROWS = NP // 128
DUMMY = N
E = 320000
ET = NP
EP = NW * ET
NVEC = NP // L
CH = 64
NCH = ET // CH

_mesh = plsc.VectorSubcoreMesh(core_axis_name="c", subcore_axis_name="s")
_sc_params = pltpu.CompilerParams(needs_layout_passes=False)


@functools.partial(
    pl.kernel, mesh=_mesh, compiler_params=_sc_params,
    out_type=(jax.ShapeDtypeStruct((NW, ET), jnp.int32),
              jax.ShapeDtypeStruct((NW, NP), jnp.float32)),
    scratch_types=[pltpu.VMEM((ET,), jnp.int32),
                   pltpu.VMEM((ET,), jnp.int32),
                   pltpu.VMEM((NP,), jnp.int32),
                   pltpu.VMEM((ET,), jnp.int32),
                   pltpu.VMEM((NP,), jnp.float32)],
)
def _sc_deg(src_hbm, dst_hbm, keep_hbm, ndst_hbm, degp_hbm,
            src_v, dst_v, keep_v, ndst_v, acc_v):
    w = lax.axis_index("s") * NC + lax.axis_index("c")
    pltpu.sync_copy(src_hbm.at[w], src_v)
    pltpu.sync_copy(dst_hbm.at[w], dst_v)
    pltpu.sync_copy(keep_hbm, keep_v)

    def zero(i, carry):
        acc_v[pl.ds(i * L, L)] = jnp.zeros((L,), jnp.float32)
        return carry
    lax.fori_loop(0, NVEC, zero, 0)

    ones = jnp.ones((L,), jnp.float32)

    def body(i, carry):
        vs = src_v[pl.ds(i * L, L)]
        vd = dst_v[pl.ds(i * L, L)]
        ks = plsc.load_gather(keep_v, [vs])
        kd = plsc.load_gather(keep_v, [vd])
        nd = jnp.where((ks > 0) & (kd > 0), vd, DUMMY)
        ndst_v[pl.ds(i * L, L)] = nd
        plsc.addupdate_scatter(acc_v, [nd], ones)
        return carry
    lax.fori_loop(0, NVEC, body, 0)

    pltpu.sync_copy(ndst_v, ndst_hbm.at[w])
    pltpu.sync_copy(acc_v, degp_hbm.at[w])


@functools.partial(
    pl.kernel, mesh=_mesh, compiler_params=_sc_params,
    out_type=jax.ShapeDtypeStruct((NC, NP, F), jnp.float32),
    scratch_types=[pltpu.VMEM((40, 128), jnp.int32),
                   pltpu.VMEM((40, 128), jnp.int32),
                   pltpu.VMEM((128, F), jnp.float32),
                   pltpu.VMEM((128, F), jnp.float32),
                   pltpu.VMEM_SHARED((NP, F), jnp.float32),
                   pltpu.SemaphoreType.DMA,
                   pltpu.SemaphoreType.DMA],
)
def _sc_rowagg(g_hbm, src_hbm, dst_hbm, zeros_hbm, part_hbm,
               src_v, dst_v, rows0_v, rows1_v, acc_sh, sem0, sem1):
    c = lax.axis_index("c")
    s = lax.axis_index("s")
    w = s * NC + c
    pltpu.sync_copy(zeros_hbm, acc_sh.at[pl.ds(s * (NP // NS), NP // NS)])
    plsc.subcore_barrier()

    dummy = zeros_hbm.at[pl.ds(0, 128)]

    def _drain(buf, sem):
        pltpu.make_async_copy(dummy, buf, sem).wait()

    for gbase, nrows in ((0, 40), (40, 39)):
        pltpu.sync_copy(src_hbm.at[w, pl.ds(gbase, nrows)],
                        src_v.at[pl.ds(0, nrows)])
        pltpu.sync_copy(dst_hbm.at[w, pl.ds(gbase, nrows)],
                        dst_v.at[pl.ds(0, nrows)])
        pltpu.async_copy(g_hbm.at[src_v.at[0]], rows0_v, sem0)

        def body(k, carry, nrows=nrows):
            e = k * 2
            _drain(rows0_v, sem0)
            pltpu.async_copy(g_hbm.at[src_v.at[e + 1]], rows1_v, sem1)
            pltpu.sync_copy(rows0_v, acc_sh.at[dst_v.at[e]], add=True)
            _drain(rows1_v, sem1)

            @pl.when(e + 2 < nrows)
            def _():
                pltpu.async_copy(g_hbm.at[src_v.at[e + 2]], rows0_v, sem0)
            pltpu.sync_copy(rows1_v, acc_sh.at[dst_v.at[e + 1]], add=True)
            return carry
        lax.fori_loop(0, nrows // 2, body, 0)
        if nrows % 2:
            _drain(rows0_v, sem0)
            pltpu.sync_copy(rows0_v, acc_sh.at[dst_v.at[nrows - 1]], add=True)

    plsc.subcore_barrier()
    stripe = pl.ds(s * (NP // NS), NP // NS)
    pltpu.sync_copy(acc_sh.at[stripe], part_hbm.at[c, stripe])


@functools.partial(
    pl.kernel, mesh=_mesh, compiler_params=_sc_params,
    out_type=jax.ShapeDtypeStruct((NW, NP), jnp.float32),
    scratch_types=[pltpu.VMEM((ET,), jnp.int32),
                   pltpu.VMEM((ET,), jnp.int32),
                   pltpu.VMEM((NP,), jnp.float32),
                   pltpu.VMEM((NP,), jnp.float32)],
)
def _sc_scalagg(gs_hbm, src_hbm, dst_hbm, sagg_hbm,
                src_v, dst_v, gs_v, acc_v):
    w = lax.axis_index("s") * NC + lax.axis_index("c")
    pltpu.sync_copy(src_hbm.at[w], src_v)
    pltpu.sync_copy(dst_hbm.at[w], dst_v)
    pltpu.sync_copy(gs_hbm, gs_v)

    def zero(i, carry):
        acc_v[pl.ds(i * L, L)] = jnp.zeros((L,), jnp.float32)
        return carry
    lax.fori_loop(0, NVEC, zero, 0)

    def body(i, carry):
        vs = src_v[pl.ds(i * L, L)]
        vd = dst_v[pl.ds(i * L, L)]
        val = plsc.load_gather(gs_v, [vs])
        plsc.addupdate_scatter(acc_v, [vd], val)
        return carry
    lax.fori_loop(0, NVEC, body, 0)

    pltpu.sync_copy(acc_v, sagg_hbm.at[w])


def _tc1_body(x_ref, w_ref, degp_ref, nm_ref, g_ref, st_ref):
    deg = jnp.sum(degp_ref[...], axis=0) + nm_ref[0, 0, :]
    dinv = jnp.where(deg > 0, 1.0 / jnp.sqrt(deg), 0.0)
    h_pre = jnp.dot(x_ref[...], w_ref[...], preferred_element_type=jnp.float32)
    g_ref[...] = h_pre * dinv[:, None]
    st_ref[...] = h_pre * (nm_ref[0, 0, :] * dinv * dinv)[:, None]


def _tc1(x, W, degp, nm3):
    return pl.pallas_call(
        _tc1_body,
        grid=(ROWS,),
        in_specs=[pl.BlockSpec((128, F), lambda i: (i, 0)),
                  pl.BlockSpec((F, F), lambda i: (0, 0)),
                  pl.BlockSpec((NW, 128), lambda i: (0, i)),
                  pl.BlockSpec((1, 1, 128), lambda i: (i, 0, 0))],
        out_specs=[pl.BlockSpec((128, F), lambda i: (i, 0)),
                   pl.BlockSpec((128, F), lambda i: (i, 0))],
        out_shape=[jax.ShapeDtypeStruct((NP, F), jnp.float32),
                   jax.ShapeDtypeStruct((NP, F), jnp.float32)],
    )(x, W, degp, nm3)


def _tc2_body(part_ref, st_ref, degp_ref, nm_ref, b_ref, ws_ref,
              h_ref, spre_ref, gs_ref):
    deg = jnp.sum(degp_ref[...], axis=0) + nm_ref[0, 0, :]
    dinv = jnp.where(deg > 0, 1.0 / jnp.sqrt(deg), 0.0)
    agg = part_ref[0] + part_ref[1]
    h = jnp.maximum(agg * dinv[:, None] + st_ref[...] + b_ref[0, 0, :][None, :], 0.0)
    h_ref[...] = h
    spre = jnp.dot(h, ws_ref[0, 0, :], preferred_element_type=jnp.float32)
    spre_ref[0, 0, :] = spre
    gs_ref[0, 0, :] = spre * dinv


def _tc2(part, st, degp, nm3, b3, ws3):
    return pl.pallas_call(
        _tc2_body,
        grid=(ROWS,),
        in_specs=[pl.BlockSpec((NC, 128, F), lambda i: (0, i, 0)),
                  pl.BlockSpec((128, F), lambda i: (i, 0)),
                  pl.BlockSpec((NW, 128), lambda i: (0, i)),
                  pl.BlockSpec((1, 1, 128), lambda i: (i, 0, 0)),
                  pl.BlockSpec((1, 1, 128), lambda i: (0, 0, 0)),
                  pl.BlockSpec((1, 1, 128), lambda i: (0, 0, 0))],
        out_specs=[pl.BlockSpec((128, F), lambda i: (i, 0)),
                   pl.BlockSpec((1, 1, 128), lambda i: (i, 0, 0)),
                   pl.BlockSpec((1, 1, 128), lambda i: (i, 0, 0))],
        out_shape=[jax.ShapeDtypeStruct((NP, F), jnp.float32),
                   jax.ShapeDtypeStruct((ROWS, 1, 128), jnp.float32),
                   jax.ShapeDtypeStruct((ROWS, 1, 128), jnp.float32)],
    )(part, st, degp, nm3, b3, ws3)


def _tc3_body(sagg_ref, degp_ref, nm_ref, spre_ref, bs_ref, score_ref):
    deg = jnp.sum(degp_ref[...], axis=0) + nm_ref[0, 0, :]
    dinv = jnp.where(deg > 0, 1.0 / jnp.sqrt(deg), 0.0)
    sa = jnp.sum(sagg_ref[...], axis=0)
    score_ref[0, 0, :] = (sa * dinv
                          + spre_ref[0, 0, :] * (nm_ref[0, 0, :] * dinv * dinv)
                          + bs_ref[0, 0, :])


def _tc3(sagg, degp, nm3, spre3, bs3):
    return pl.pallas_call(
        _tc3_body,
        grid=(ROWS,),
        in_specs=[pl.BlockSpec((NW, 128), lambda i: (0, i)),
                  pl.BlockSpec((NW, 128), lambda i: (0, i)),
                  pl.BlockSpec((1, 1, 128), lambda i: (i, 0, 0)),
                  pl.BlockSpec((1, 1, 128), lambda i: (i, 0, 0)),
                  pl.BlockSpec((1, 1, 128), lambda i: (0, 0, 0))],
        out_specs=pl.BlockSpec((1, 1, 128), lambda i: (i, 0, 0)),
        out_shape=jax.ShapeDtypeStruct((ROWS, 1, 128), jnp.float32),
    )(sagg, degp, nm3, spre3, bs3)


def _tc4_body(h_ref, score_ref, keep_ref, batch_ref, lo_ref, hi_ref,
              hp_ref, sums_ref, cnts_ref, maxs_ref):
    i = pl.program_id(0)
    th = jnp.tanh(score_ref[0, 0, :])
    hpb = h_ref[...] * th[:, None]
    hp_ref[...] = hpb
    keep = keep_ref[0, 0, :]
    kb = jnp.where(keep > 0, batch_ref[0, 0, :], G)

    onehot = (kb[:, None] == lax.broadcasted_iota(jnp.int32, (128, G), 1)
              ).astype(jnp.float32)
    psum = lax.dot_general(onehot, hpb, (((0,), (0,)), ((), ())),
                           preferred_element_type=jnp.float32)
    pcnt = jnp.sum(onehot, axis=0)

    neg = jnp.float32(-3.0e38)

    @pl.when(i == 0)
    def _():
        sums_ref[...] = jnp.zeros((G, F), jnp.float32)
        cnts_ref[...] = jnp.zeros((G, 128), jnp.float32)
        maxs_ref[...] = jnp.full((G, F), neg, jnp.float32)

    sums_ref[...] += psum
    cnts_ref[...] += pcnt[:, None]

    lo = lo_ref[0, 0, 0]
    hi = jnp.minimum(hi_ref[0, 0, 0], G - 1)

    def body(g, carry):
        mf = (kb == g).astype(jnp.float32)[:, None]
        mg = jnp.max(hpb * mf + neg * (1.0 - mf), axis=0)
        maxs_ref[pl.ds(g, 1), :] = jnp.maximum(maxs_ref[pl.ds(g, 1), :],
                                               mg[None, :])
        return carry
    lax.fori_loop(lo, hi + 1, body, 0)


def _tc4(h, score3, keep3, batch3, lo3b, hi3b):
    return pl.pallas_call(
        _tc4_body,
        grid=(ROWS,),
        in_specs=[pl.BlockSpec((128, F), lambda i: (i, 0)),
                  pl.BlockSpec((1, 1, 128), lambda i: (i, 0, 0)),
                  pl.BlockSpec((1, 1, 128), lambda i: (i, 0, 0)),
                  pl.BlockSpec((1, 1, 128), lambda i: (i, 0, 0)),
                  pl.BlockSpec((1, 1, 128), lambda i: (i, 0, 0)),
                  pl.BlockSpec((1, 1, 128), lambda i: (i, 0, 0))],
        out_specs=[pl.BlockSpec((128, F), lambda i: (i, 0)),
                   pl.BlockSpec((G, F), lambda i: (0, 0)),
                   pl.BlockSpec((G, 128), lambda i: (0, 0)),
                   pl.BlockSpec((G, F), lambda i: (0, 0))],
        out_shape=[jax.ShapeDtypeStruct((NP, F), jnp.float32),
                   jax.ShapeDtypeStruct((G, F), jnp.float32),
                   jax.ShapeDtypeStruct((G, 128), jnp.float32),
                   jax.ShapeDtypeStruct((G, F), jnp.float32)],
    )(h, score3, keep3, batch3, lo3b, hi3b)


def _tc5_body(xs_ref, wl_ref, bl_ref, out_ref):
    out_ref[...] = jnp.maximum(
        jnp.dot(xs_ref[...], wl_ref[...], preferred_element_type=jnp.float32)
        + bl_ref[0][None, :], 0.0)


def _tc5(xs, Wl, bl2):
    return pl.pallas_call(
        _tc5_body,
        in_specs=[pl.BlockSpec((G, 2 * F), lambda: (0, 0)),
                  pl.BlockSpec((2 * F, F), lambda: (0, 0)),
                  pl.BlockSpec((8, F), lambda: (0, 0))],
        out_specs=pl.BlockSpec((G, F), lambda: (0, 0)),
        out_shape=jax.ShapeDtypeStruct((G, F), jnp.float32),
    )(xs, Wl, bl2)


def _tcrank_body(sa_ref, ba_ref, va_ref, st_ref, bt_ref, lo_ref, hi_ref,
                 rank_ref):
    i = pl.program_id(0)
    rs = st_ref[0, 0, :][None, :]
    rb = bt_ref[0, 0, :][None, :]
    ridx = lax.broadcasted_iota(jnp.int32, (1, 128), 1) + i * 128
    lo = lo_ref[0, 0, 0]
    hi = hi_ref[0, 0, 0]

    def body(c, acc):
        cs = sa_ref[c, 0, :][:, None]
        cb = ba_ref[c, 0, :][:, None]
        cv = va_ref[c, 0, :][:, None]
        cidx = lax.broadcasted_iota(jnp.int32, (128, 1), 0) + c * 128
        beats = (cs > rs) | ((cs == rs) & (cidx < ridx))
        contrib = jnp.where((cb == rb) & (cv > 0) & beats, 1.0, 0.0)
        return acc + jnp.sum(contrib, axis=0)
    rank = lax.fori_loop(lo, hi + 1, body, jnp.zeros((128,), jnp.float32))
    rank_ref[0, 0, :] = rank


def _tc_rank(score3, batch3f, valid3, lo3, hi3):
    whole = pl.BlockSpec((ROWS, 1, 128), lambda i: (0, 0, 0))
    tile = pl.BlockSpec((1, 1, 128), lambda i: (i, 0, 0))
    return pl.pallas_call(
        _tcrank_body,
        grid=(ROWS,),
        in_specs=[whole, whole, whole, tile, tile, tile, tile],
        out_specs=tile,
        out_shape=jax.ShapeDtypeStruct((ROWS, 1, 128), jnp.float32),
    )(score3, batch3f, valid3, score3, batch3f, lo3, hi3)


def _topk_keep_pallas(score3, nm, batch_pad, batch3f, tile_min, tile_max):
    valid = nm > 0
    bmask = jnp.where(valid, batch_pad, G)
    counts = jnp.bincount(bmask, length=G + 1)
    k = jnp.ceil(RATIO * counts).astype(jnp.int32).at[G].set(0)
    lo_t = jnp.searchsorted(tile_max, tile_min, side='left').astype(jnp.int32)
    hi_t = (jnp.searchsorted(tile_min, tile_max, side='right') - 1).astype(jnp.int32)
    lo3 = jnp.broadcast_to(lo_t[:, None, None], (ROWS, 1, 128))
    hi3 = jnp.broadcast_to(hi_t[:, None, None], (ROWS, 1, 128))
    valid3 = nm.reshape(ROWS, 1, 128)
    rank = _tc_rank(score3, batch3f, valid3, lo3, hi3).reshape(NP)
    return valid & (rank < k[bmask].astype(jnp.float32))


def kernel(x, edge_index, batch, W1, b1, Ws1, bs1, W2, b2, Ws2, bs2,
           W3, b3, Ws3, bs3, Wl, bl):
    f32 = jnp.float32
    xp = jnp.zeros((NP, F), f32).at[:N, :].set(x)
    batch_pad = jnp.concatenate([batch.astype(jnp.int32),
                                 jnp.full((NP - N,), G, jnp.int32)])
    batch3 = batch_pad.reshape(ROWS, 1, 128)
    batch3f = batch_pad.astype(f32).reshape(ROWS, 1, 128)
    bt2 = batch_pad.reshape(ROWS, 128)
    tile_min = bt2.min(axis=1).astype(jnp.int32)
    tile_max = bt2.max(axis=1).astype(jnp.int32)
    lo3b = jnp.broadcast_to(tile_min[:, None, None], (ROWS, 1, 128))
    hi3b = jnp.broadcast_to(tile_max[:, None, None], (ROWS, 1, 128))
    src = jnp.concatenate([edge_index[0].astype(jnp.int32),
                           jnp.zeros((EP - E,), jnp.int32)])
    dst = jnp.concatenate([edge_index[1].astype(jnp.int32),
                           jnp.full((EP - E,), DUMMY, jnp.int32)])
    src2 = src.reshape(NW, ET)
    src3 = src.reshape(NW, ROWS, 128)
    dst2 = dst.reshape(NW, ET)

    zeros_rows = jnp.zeros((NP // NS, F), f32)
    keep_i = jnp.concatenate([jnp.ones((N,), jnp.int32),
                              jnp.zeros((NP - N,), jnp.int32)])
    nm = keep_i.astype(f32)
    dst_cur2 = dst2
    xcur = xp
    xs = None
    for (W, b, Ws, bs) in ((W1, b1, Ws1, bs1), (W2, b2, Ws2, bs2),
                           (W3, b3, Ws3, bs3)):
        nm3 = nm.reshape(ROWS, 1, 128)
        b3r = b.reshape(1, 1, 128)
        ws3r = Ws.reshape(1, 1, 128)
        bs3r = jnp.broadcast_to(bs.reshape(1, 1, 1), (1, 1, 128))

        ndst2, degp = _sc_deg(src2, dst_cur2, keep_i)
        dst_cur2 = ndst2
        dst3 = ndst2.reshape(NW, ROWS, 128)

        g, st = _tc1(xcur, W, degp, nm3)
        part = _sc_rowagg(g, src3, dst3, zeros_rows)
        h, spre3, gs3 = _tc2(part, st, degp, nm3, b3r, ws3r)
        sagg = _sc_scalagg(gs3.reshape(NP), src2, ndst2)
        score3 = _tc3(sagg, degp, nm3, spre3, bs3r)

        keep = _topk_keep_pallas(score3, nm, batch_pad, batch3f,
                                 tile_min, tile_max)
        keep_i = keep.astype(jnp.int32)
        keep3 = keep_i.astype(f32).reshape(ROWS, 1, 128)

        hp, sums, cnts, maxs = _tc4(h, score3, keep3, batch3, lo3b, hi3b)
        cnt = cnts[:, 0]
        gmp = jnp.where(cnt[:, None] > 0, maxs, 0.0)
        gap = sums / jnp.maximum(cnt, 1.0)[:, None]
        ro = jnp.concatenate([gmp, gap], axis=1)
        xs = ro if xs is None else xs + ro

        nm = keep_i.astype(f32)
        xcur = hp

    bl2 = jnp.broadcast_to(bl[None, :], (8, F))
    return _tc5(xs, Wl, bl2)
